# Initial kernel scaffold; baseline (speedup 1.0000x reference)
import jax
import jax.numpy as jnp
from jax.experimental import pallas as pl


def kernel(x, edge_index, edge_attr, W1, as1, ad1, We1, ae1, b1, W2, as2, ad2, We2, ae2, b2, W3, as3, ad3, We3, ae3, b3, W4, b4):
    raise NotImplementedError("write your pallas kernel here")


# speedup vs baseline: 19.4103x; 19.4103x over previous
import functools
import jax
import jax.numpy as jnp
from jax import lax
from jax.experimental import pallas as pl
from jax.experimental.pallas import tpu as pltpu, tpu_sc as plsc

N = 10000
E = 320000
NEG = 0.2
NC = 2
NS = 16
NW = NC * NS
EPW = E // NW
ROWS_PW = N // NS
CH = 400
NCHUNK = EPW // CH


def _lrelu(v):
    return jnp.where(v > 0, v, NEG * v)


_EB = E // 128


def _tc_et_body(ea_ref, we1_ref, ae1_ref, we2_ref, ae2_ref, we3_ref, ae3_ref,
                et4_ref, e1_ref, e2_ref, e3_ref, etmax_ref):
    ea = ea_ref[...]
    ves = []
    for we_ref, ae_ref in ((we1_ref, ae1_ref), (we2_ref, ae2_ref), (we3_ref, ae3_ref)):
        ves.append(we_ref[...] @ ae_ref[...])

    p_id = lax.broadcasted_iota(jnp.int32, (512, 1), 0)
    k_id = lax.broadcasted_iota(jnp.int32, (512, 4), 1)
    P4 = (p_id % 4 == k_id).astype(jnp.float32)

    c_id = lax.broadcasted_iota(jnp.int32, (512, 128), 1)
    blk128 = (p_id // 4 == c_id).astype(jnp.float32)
    maxes = []
    for ve, out_ref in zip(ves, (e1_ref, e2_ref, e3_ref)):
        Bj = (P4 @ ve[:, None]) * blk128
        ej = ea @ Bj
        out_ref[...] = ej
        maxes.append(ej.max())
    etmax_ref[...] = jnp.stack(maxes)[None, :]

    q_id = lax.broadcasted_iota(jnp.int32, (512, 512), 1)
    j_id = q_id % 4
    Q4 = jnp.stack([(j_id[0] == j).astype(jnp.float32) for j in range(3)], axis=1)
    V = jnp.stack(ves, axis=1)
    blk44 = (p_id // 4 == q_id // 4).astype(jnp.float32)
    B4 = (P4 @ V @ Q4.T) * blk44
    pat = (j_id[0] == 3).astype(jnp.float32)
    et4_ref[...] = ea @ B4 + pat[None, :]


def _tc_et(ea_fl, We1, ae1, We2, ae2, We3, ae3):
    return pl.pallas_call(
        _tc_et_body,
        out_shape=(
            jax.ShapeDtypeStruct((_EB, 512), jnp.float32),
            jax.ShapeDtypeStruct((_EB, 128), jnp.float32),
            jax.ShapeDtypeStruct((_EB, 128), jnp.float32),
            jax.ShapeDtypeStruct((_EB, 128), jnp.float32),
            jax.ShapeDtypeStruct((1, 3), jnp.float32),
        ),
    )(ea_fl, We1, ae1, We2, ae2, We3, ae3)


def _tc_l1_body(x_ref, w_ref, as_ref, ad_ref, n0_ref, n1_ref,
                h_ref, s_ref, d_ref, sdmax_ref, e1_ref, e2_ref, e3_ref,
                etlmax_ref):
    h = x_ref[...] @ w_ref[...]
    h_ref[...] = h
    s = h @ as_ref[...]
    d = h @ ad_ref[...]
    s_ref[...] = s
    d_ref[...] = d
    sdmax_ref[...] = jnp.stack([s.max(), d.max()])[None, :]
    nacc = n0_ref[...] + n1_ref[...]
    deg = jnp.clip(nacc[:, 3], 1.0, None)
    maxes = []
    for j, er in enumerate((e1_ref, e2_ref, e3_ref)):
        etl = nacc[:, j] / deg
        er[...] = etl
        maxes.append(etl.max())
    etlmax_ref[...] = jnp.stack(maxes)[None, :]


def _tc_l1(x, W1, as1, ad1, nacc0, nacc1):
    return pl.pallas_call(
        _tc_l1_body,
        out_shape=(
            jax.ShapeDtypeStruct((N, 32), jnp.float32),
            jax.ShapeDtypeStruct((N,), jnp.float32),
            jax.ShapeDtypeStruct((N,), jnp.float32),
            jax.ShapeDtypeStruct((1, 2), jnp.float32),
            jax.ShapeDtypeStruct((N,), jnp.float32),
            jax.ShapeDtypeStruct((N,), jnp.float32),
            jax.ShapeDtypeStruct((N,), jnp.float32),
            jax.ShapeDtypeStruct((1, 3), jnp.float32),
        ),
    )(x, W1, as1, ad1, nacc0, nacc1)


def _combine(a0_ref, a1_ref, h_ref, s_ref, d_ref, etl_ref, m_ref, b_ref, dh_prev):
    al = _lrelu(s_ref[...] + d_ref[...] + etl_ref[...])
    exl = jnp.exp(al - m_ref[0])[:, None]
    accd = a0_ref[...] + a1_ref[...]
    acc = accd[:, :dh_prev]
    den = accd[:, dh_prev:dh_prev + 1]
    out = (acc + exl * h_ref[...]) / (den + exl) + b_ref[...][None, :]
    return jnp.maximum(out, 0.0)


def _tc_lj_body(dh_prev, a0_ref, a1_ref, h_ref, s_ref, d_ref, etl_ref, m_ref,
                b_ref, w_ref, as_ref, ad_ref,
                hn_ref, sn_ref, dn_ref, sdmax_ref):
    out = _combine(a0_ref, a1_ref, h_ref, s_ref, d_ref, etl_ref, m_ref, b_ref,
                   dh_prev)
    hn = out @ w_ref[...]
    hn_ref[...] = hn
    s = hn @ as_ref[...]
    d = hn @ ad_ref[...]
    sn_ref[...] = s
    dn_ref[...] = d
    sdmax_ref[...] = jnp.stack([s.max(), d.max()])[None, :]


def _tc_lj(dh_prev, dh, accD, h_prev, s, d, etl_col, M, b_prev, W, a_s, a_d):
    m2 = jnp.full((1,), M, jnp.float32)
    return pl.pallas_call(
        functools.partial(_tc_lj_body, dh_prev),
        out_shape=(
            jax.ShapeDtypeStruct((N, dh), jnp.float32),
            jax.ShapeDtypeStruct((N,), jnp.float32),
            jax.ShapeDtypeStruct((N,), jnp.float32),
            jax.ShapeDtypeStruct((1, 2), jnp.float32),
        ),
        in_specs=[pl.BlockSpec(memory_space=pltpu.VMEM)] * 6
        + [pl.BlockSpec(memory_space=pltpu.SMEM)]
        + [pl.BlockSpec(memory_space=pltpu.VMEM)] * 4,
    )(accD[0], accD[1], h_prev, s, d, etl_col, m2, b_prev, W, a_s, a_d)


def _tc_l4_body(a0_ref, a1_ref, h_ref, s_ref, d_ref, etl_ref, m_ref,
                b3_ref, w4_ref, b4_ref, y_ref):
    out = _combine(a0_ref, a1_ref, h_ref, s_ref, d_ref, etl_ref, m_ref, b3_ref, 64)
    y_ref[...] = out @ w4_ref[...] + b4_ref[...][None, :]


def _tc_l4(accD, h3, s3, d3, etl3, M3, b3, W4, b4):
    m2 = jnp.full((1,), M3, jnp.float32)
    return pl.pallas_call(
        _tc_l4_body,
        out_shape=jax.ShapeDtypeStruct((N, 512), jnp.float32),
        in_specs=[pl.BlockSpec(memory_space=pltpu.VMEM)] * 6
        + [pl.BlockSpec(memory_space=pltpu.SMEM)]
        + [pl.BlockSpec(memory_space=pltpu.VMEM)] * 3,
    )(accD[0], accD[1], h3, s3, d3, etl3, m2, b3, W4, b4)


_MESH = plsc.VectorSubcoreMesh(core_axis_name="c", subcore_axis_name="s")


ZROWS = 1000
ZTILES = N // ZROWS


def _zero_spmem_slice(acc_sh, zeros_hbm, sid):
    @pl.when(sid < ZTILES)
    def _():
        base = sid * ZROWS
        pltpu.sync_copy(zeros_hbm.at[pl.ds(base, ZROWS)],
                        acc_sh.at[pl.ds(base, ZROWS)])


def _sc_pre_body(et4_hbm, dst_hbm, z_hbm, out_hbm,
                 et4_v, dst_v, acc_sh, sem):
    cid = lax.axis_index("c")
    sid = lax.axis_index("s")
    wid = cid * NS + sid

    _zero_spmem_slice(acc_sh, z_hbm, sid)
    plsc.subcore_barrier()

    def chunk(i, _):
        base = wid * EPW + i * CH
        pltpu.sync_copy(dst_hbm.at[pl.ds(base, CH)], dst_v)
        pltpu.sync_copy(et4_hbm.at[pl.ds(base, CH)], et4_v)
        pltpu.sync_copy(et4_v, acc_sh.at[dst_v], add=True)
        return 0

    lax.fori_loop(0, NCHUNK, chunk, 0)
    plsc.subcore_barrier()

    @pl.when(sid < ZTILES)
    def _():
        pltpu.sync_copy(acc_sh.at[pl.ds(sid * ZROWS, ZROWS)],
                        out_hbm.at[cid, pl.ds(sid * ZROWS, ZROWS)])


@functools.partial(
    pl.kernel,
    out_type=jax.ShapeDtypeStruct((NC, N, 4), jnp.float32),
    mesh=_MESH,
    compiler_params=pltpu.CompilerParams(needs_layout_passes=False, use_tc_tiling_on_sc=False),
    scratch_types=[
        pltpu.VMEM((CH, 4), jnp.float32),
        pltpu.VMEM((CH,), jnp.int32),
        pltpu.VMEM_SHARED((N, 4), jnp.float32),
        pltpu.SemaphoreType.DMA,
    ],
)
def _sc_pre(et4_hbm, dst_hbm, z_hbm, out_hbm, *scratch):
    _sc_pre_body(et4_hbm, dst_hbm, z_hbm, out_hbm, *scratch)


def _sc_layer_body(dh, src_hbm, dst_hbm, et_hbm, s_hbm, d_hbm, m_hbm, h_hbm,
                   z_hbm, out_hbm, s_v, d_v, m_v, src_v, dst_v, et_v, ex_v,
                   rowsh_v, rows_v, acc_sh, sem):
    cid = lax.axis_index("c")
    sid = lax.axis_index("s")
    wid = cid * NS + sid

    pltpu.sync_copy(s_hbm, s_v)
    pltpu.sync_copy(d_hbm, d_v)
    pltpu.sync_copy(m_hbm, m_v)
    _zero_spmem_slice(acc_sh, z_hbm, sid)
    plsc.subcore_barrier()

    mvec = m_v[...]
    unit = jnp.where(lax.iota(jnp.int32, 16) == 0, 1.0, 0.0)

    def chunk(i, _):
        base = wid * EPW + i * CH
        pltpu.sync_copy(src_hbm.at[pl.ds(base, CH)], src_v)
        pltpu.sync_copy(dst_hbm.at[pl.ds(base, CH)], dst_v)
        pltpu.sync_copy(et_hbm.at[pl.ds(base, CH)], et_v)
        pltpu.async_copy(h_hbm.at[src_v], rowsh_v, sem).wait()

        def attn(r, _):
            o = pl.multiple_of(r * 16, 16)
            srcv = src_v[pl.ds(o, 16)]
            dstv = dst_v[pl.ds(o, 16)]
            sv = plsc.load_gather(s_v, [srcv])
            dv = plsc.load_gather(d_v, [dstv])
            al = _lrelu(sv + dv + et_v[pl.ds(o, 16)])
            ex_v[pl.ds(o, 16)] = jnp.exp(al - mvec)
            return 0

        lax.fori_loop(0, CH // 16, attn, 0)

        def scale(e, _):
            x = plsc.load_gather(ex_v, [jnp.full((16,), e, jnp.int32)])
            for k in range(dh // 16):
                rows_v[e, pl.ds(k * 16, 16)] = rowsh_v[e, pl.ds(k * 16, 16)] * x
            rows_v[e, pl.ds(dh, 16)] = unit * x
            return 0

        lax.fori_loop(0, CH, scale, 0)
        pltpu.sync_copy(rows_v, acc_sh.at[dst_v], add=True)
        return 0

    lax.fori_loop(0, NCHUNK, chunk, 0)
    plsc.subcore_barrier()

    @pl.when(sid < ZTILES)
    def _():
        pltpu.sync_copy(acc_sh.at[pl.ds(sid * ZROWS, ZROWS)],
                        out_hbm.at[cid, pl.ds(sid * ZROWS, ZROWS)])


def _make_sc_layer(dh):
    dhp = dh + 16

    @functools.partial(
        pl.kernel,
        out_type=jax.ShapeDtypeStruct((NC, N, dhp), jnp.float32),
        mesh=_MESH,
        compiler_params=pltpu.CompilerParams(needs_layout_passes=False, use_tc_tiling_on_sc=False),
        scratch_types=[
            pltpu.VMEM((N,), jnp.float32),
            pltpu.VMEM((N,), jnp.float32),
            pltpu.VMEM((16,), jnp.float32),
            pltpu.VMEM((CH,), jnp.int32),
            pltpu.VMEM((CH,), jnp.int32),
            pltpu.VMEM((CH,), jnp.float32),
            pltpu.VMEM((CH,), jnp.float32),
            pltpu.VMEM((CH, dh), jnp.float32),
            pltpu.VMEM((CH, dhp), jnp.float32),
            pltpu.VMEM_SHARED((N, dhp), jnp.float32),
            pltpu.SemaphoreType.DMA,
        ],
    )
    def sc_layer(src_hbm, dst_hbm, et_hbm, s_hbm, d_hbm, m_hbm, h_hbm,
                 z_hbm, out_hbm, *scratch):
        _sc_layer_body(dh, src_hbm, dst_hbm, et_hbm, s_hbm, d_hbm, m_hbm,
                       h_hbm, z_hbm, out_hbm, *scratch)

    return sc_layer


_sc_layer32 = _make_sc_layer(32)
_sc_layer64 = _make_sc_layer(64)


def kernel(x, edge_index, edge_attr, W1, as1, ad1, We1, ae1, b1,
           W2, as2, ad2, We2, ae2, b2, W3, as3, ad3, We3, ae3, b3, W4, b4):
    src = edge_index[0]
    dst = edge_index[1]

    z4 = jnp.zeros((N, 4), jnp.float32)
    z48 = jnp.zeros((N, 48), jnp.float32)
    z80 = jnp.zeros((N, 80), jnp.float32)

    ea_fl = edge_attr.reshape(_EB, 512)
    et4_2d, e1_2d, e2_2d, e3_2d, etmax = _tc_et(ea_fl, We1, ae1, We2, ae2, We3, ae3)
    et4 = et4_2d.reshape(E, 4)
    ett = (e1_2d.reshape(E), e2_2d.reshape(E), e3_2d.reshape(E))
    naccD = _sc_pre(et4, dst, z4)
    h1, s1, d1, sdmax1, etl1, etl2, etl3, etlmax = _tc_l1(
        x, W1, as1, ad1, naccD[0], naccD[1])

    emax = jnp.maximum(etmax[0], etlmax[0])

    M1 = sdmax1[0, 0] + sdmax1[0, 1] + emax[0]
    m1 = jnp.full((16,), M1, jnp.float32)
    accD1 = _sc_layer32(src, dst, ett[0], s1, d1, m1, h1, z48)

    h2, s2, d2, sdmax2 = _tc_lj(32, 64, accD1, h1, s1, d1, etl1, M1,
                                b1, W2, as2, ad2)
    M2 = sdmax2[0, 0] + sdmax2[0, 1] + emax[1]
    m2 = jnp.full((16,), M2, jnp.float32)
    accD2 = _sc_layer64(src, dst, ett[1], s2, d2, m2, h2, z80)

    h3, s3, d3, sdmax3 = _tc_lj(64, 64, accD2, h2, s2, d2, etl2, M2,
                                b2, W3, as3, ad3)
    M3 = sdmax3[0, 0] + sdmax3[0, 1] + emax[2]
    m3 = jnp.full((16,), M3, jnp.float32)
    accD3 = _sc_layer64(src, dst, ett[2], s3, d3, m3, h3, z80)

    return _tc_l4(accD3, h3, s3, d3, etl3, M3, b3, W4, b4)


# speedup vs baseline: 23.8129x; 1.2268x over previous
import functools
import jax
import jax.numpy as jnp
from jax import lax
from jax.experimental import pallas as pl
from jax.experimental.pallas import tpu as pltpu, tpu_sc as plsc

N = 10000
E = 320000
NEG = 0.2
NC = 2
NS = 16
NW = NC * NS
EPW = E // NW
ROWS_PW = N // NS
CH = 400
NCHUNK = EPW // CH


def _lrelu(v):
    return jnp.where(v > 0, v, NEG * v)


_EB = E // 128


def _tc_et_body(ea_ref, we1_ref, ae1_ref, we2_ref, ae2_ref, we3_ref, ae3_ref,
                et4_ref, e1_ref, e2_ref, e3_ref, etmax_ref):
    ea = ea_ref[...]
    ves = []
    for we_ref, ae_ref in ((we1_ref, ae1_ref), (we2_ref, ae2_ref), (we3_ref, ae3_ref)):
        ves.append(we_ref[...] @ ae_ref[...])

    p_id = lax.broadcasted_iota(jnp.int32, (512, 1), 0)
    k_id = lax.broadcasted_iota(jnp.int32, (512, 4), 1)
    P4 = (p_id % 4 == k_id).astype(jnp.float32)

    c_id = lax.broadcasted_iota(jnp.int32, (512, 128), 1)
    blk128 = (p_id // 4 == c_id).astype(jnp.float32)
    maxes = []
    for ve, out_ref in zip(ves, (e1_ref, e2_ref, e3_ref)):
        Bj = (P4 @ ve[:, None]) * blk128
        ej = ea @ Bj
        out_ref[...] = ej
        maxes.append(ej.max())
    etmax_ref[...] = jnp.stack(maxes)[None, :]

    q_id = lax.broadcasted_iota(jnp.int32, (512, 512), 1)
    j_id = q_id % 4
    Q4 = jnp.stack([(j_id[0] == j).astype(jnp.float32) for j in range(3)], axis=1)
    V = jnp.stack(ves, axis=1)
    blk44 = (p_id // 4 == q_id // 4).astype(jnp.float32)
    B4 = (P4 @ V @ Q4.T) * blk44
    pat = (j_id[0] == 3).astype(jnp.float32)
    et4_ref[...] = ea @ B4 + pat[None, :]


def _tc_et(ea_fl, We1, ae1, We2, ae2, We3, ae3):
    return pl.pallas_call(
        _tc_et_body,
        out_shape=(
            jax.ShapeDtypeStruct((_EB, 512), jnp.float32),
            jax.ShapeDtypeStruct((_EB, 128), jnp.float32),
            jax.ShapeDtypeStruct((_EB, 128), jnp.float32),
            jax.ShapeDtypeStruct((_EB, 128), jnp.float32),
            jax.ShapeDtypeStruct((1, 3), jnp.float32),
        ),
    )(ea_fl, We1, ae1, We2, ae2, We3, ae3)


def _tc_l1_body(x_ref, w_ref, as_ref, ad_ref, n0_ref, n1_ref,
                h_ref, s_ref, d_ref, sdmax_ref, e1_ref, e2_ref, e3_ref,
                etlmax_ref):
    h = x_ref[...] @ w_ref[...]
    h_ref[...] = h
    s = h @ as_ref[...]
    d = h @ ad_ref[...]
    s_ref[...] = s
    d_ref[...] = d
    sdmax_ref[...] = jnp.stack([s.max(), d.max()])[None, :]
    nacc = n0_ref[...] + n1_ref[...]
    deg = jnp.clip(nacc[:, 3], 1.0, None)
    maxes = []
    for j, er in enumerate((e1_ref, e2_ref, e3_ref)):
        etl = nacc[:, j] / deg
        er[...] = etl
        maxes.append(etl.max())
    etlmax_ref[...] = jnp.stack(maxes)[None, :]


def _tc_l1(x, W1, as1, ad1, nacc0, nacc1):
    return pl.pallas_call(
        _tc_l1_body,
        out_shape=(
            jax.ShapeDtypeStruct((N, 32), jnp.float32),
            jax.ShapeDtypeStruct((N,), jnp.float32),
            jax.ShapeDtypeStruct((N,), jnp.float32),
            jax.ShapeDtypeStruct((1, 2), jnp.float32),
            jax.ShapeDtypeStruct((N,), jnp.float32),
            jax.ShapeDtypeStruct((N,), jnp.float32),
            jax.ShapeDtypeStruct((N,), jnp.float32),
            jax.ShapeDtypeStruct((1, 3), jnp.float32),
        ),
    )(x, W1, as1, ad1, nacc0, nacc1)


def _combine(a0_ref, a1_ref, n0_ref, n1_ref, h_ref, s_ref, d_ref, etl_ref,
             m_ref, b_ref):
    al = _lrelu(s_ref[...] + d_ref[...] + etl_ref[...])
    exl = jnp.exp(al - m_ref[0])[:, None]
    acc = a0_ref[...] + a1_ref[...]
    den = (n0_ref[...] + n1_ref[...])[:, 0:1]
    out = (acc + exl * h_ref[...]) / (den + exl) + b_ref[...][None, :]
    return jnp.maximum(out, 0.0)


def _tc_lj_body(a0_ref, a1_ref, n0_ref, n1_ref, h_ref, s_ref, d_ref, etl_ref,
                m_ref, b_ref, w_ref, as_ref, ad_ref,
                hn_ref, sn_ref, dn_ref, sdmax_ref):
    out = _combine(a0_ref, a1_ref, n0_ref, n1_ref, h_ref, s_ref, d_ref,
                   etl_ref, m_ref, b_ref)
    hn = out @ w_ref[...]
    hn_ref[...] = hn
    s = hn @ as_ref[...]
    d = hn @ ad_ref[...]
    sn_ref[...] = s
    dn_ref[...] = d
    sdmax_ref[...] = jnp.stack([s.max(), d.max()])[None, :]


def _tc_lj(dh, accD, h_prev, s, d, etl_col, M, b_prev, W, a_s, a_d):
    m2 = jnp.full((1,), M, jnp.float32)
    acc, den = accD
    return pl.pallas_call(
        _tc_lj_body,
        out_shape=(
            jax.ShapeDtypeStruct((N, dh), jnp.float32),
            jax.ShapeDtypeStruct((N,), jnp.float32),
            jax.ShapeDtypeStruct((N,), jnp.float32),
            jax.ShapeDtypeStruct((1, 2), jnp.float32),
        ),
        in_specs=[pl.BlockSpec(memory_space=pltpu.VMEM)] * 8
        + [pl.BlockSpec(memory_space=pltpu.SMEM)]
        + [pl.BlockSpec(memory_space=pltpu.VMEM)] * 4,
    )(acc[0], acc[1], den[0], den[1], h_prev, s, d, etl_col, m2, b_prev,
      W, a_s, a_d)


def _tc_l4_body(a0_ref, a1_ref, n0_ref, n1_ref, h_ref, s_ref, d_ref, etl_ref,
                m_ref, b3_ref, w4_ref, b4_ref, y_ref):
    out = _combine(a0_ref, a1_ref, n0_ref, n1_ref, h_ref, s_ref, d_ref,
                   etl_ref, m_ref, b3_ref)
    y_ref[...] = out @ w4_ref[...] + b4_ref[...][None, :]


def _tc_l4(accD, h3, s3, d3, etl3, M3, b3, W4, b4):
    m2 = jnp.full((1,), M3, jnp.float32)
    acc, den = accD
    return pl.pallas_call(
        _tc_l4_body,
        out_shape=jax.ShapeDtypeStruct((N, 512), jnp.float32),
        in_specs=[pl.BlockSpec(memory_space=pltpu.VMEM)] * 8
        + [pl.BlockSpec(memory_space=pltpu.SMEM)]
        + [pl.BlockSpec(memory_space=pltpu.VMEM)] * 3,
    )(acc[0], acc[1], den[0], den[1], h3, s3, d3, etl3, m2, b3, W4, b4)


_MESH = plsc.VectorSubcoreMesh(core_axis_name="c", subcore_axis_name="s")


ZROWS = 1000
ZTILES = N // ZROWS


def _zero_spmem_slice(acc_sh, zeros_hbm, sid):
    @pl.when(sid < ZTILES)
    def _():
        base = sid * ZROWS
        pltpu.sync_copy(zeros_hbm.at[pl.ds(base, ZROWS)],
                        acc_sh.at[pl.ds(base, ZROWS)])


def _sc_pre_body(et4_hbm, dst_hbm, z_hbm, out_hbm,
                 et4_v, dst_v, acc_sh, sem):
    cid = lax.axis_index("c")
    sid = lax.axis_index("s")
    wid = cid * NS + sid

    _zero_spmem_slice(acc_sh, z_hbm, sid)
    plsc.subcore_barrier()

    def chunk(i, _):
        base = wid * EPW + i * CH
        pltpu.sync_copy(dst_hbm.at[pl.ds(base, CH)], dst_v)
        pltpu.sync_copy(et4_hbm.at[pl.ds(base, CH)], et4_v)
        pltpu.sync_copy(et4_v, acc_sh.at[dst_v], add=True)
        return 0

    lax.fori_loop(0, NCHUNK, chunk, 0)
    plsc.subcore_barrier()

    @pl.when(sid < ZTILES)
    def _():
        pltpu.sync_copy(acc_sh.at[pl.ds(sid * ZROWS, ZROWS)],
                        out_hbm.at[cid, pl.ds(sid * ZROWS, ZROWS)])


@functools.partial(
    pl.kernel,
    out_type=jax.ShapeDtypeStruct((NC, N, 4), jnp.float32),
    mesh=_MESH,
    compiler_params=pltpu.CompilerParams(needs_layout_passes=False, use_tc_tiling_on_sc=False),
    scratch_types=[
        pltpu.VMEM((CH, 4), jnp.float32),
        pltpu.VMEM((CH,), jnp.int32),
        pltpu.VMEM_SHARED((N, 4), jnp.float32),
        pltpu.SemaphoreType.DMA,
    ],
)
def _sc_pre(et4_hbm, dst_hbm, z_hbm, out_hbm, *scratch):
    _sc_pre_body(et4_hbm, dst_hbm, z_hbm, out_hbm, *scratch)


def _sc_layer_body(dh, src_hbm, dst_hbm, et_hbm, s_hbm, d_hbm, m_hbm, h_hbm,
                   zacc_hbm, zden_hbm, acc_hbm, den_hbm, s_v, d_v, m_v,
                   src_a, dst_a, et_a, rowsh_a, gsem_a,
                   src_b, dst_b, et_b, rowsh_b, gsem_b,
                   ex_v, exrows_v, acc_sh, den_sh):
    cid = lax.axis_index("c")
    sid = lax.axis_index("s")
    wid = cid * NS + sid

    pltpu.sync_copy(s_hbm, s_v)
    pltpu.sync_copy(d_hbm, d_v)
    pltpu.sync_copy(m_hbm, m_v)
    _zero_spmem_slice(acc_sh, zacc_hbm, sid)
    _zero_spmem_slice(den_sh, zden_hbm, sid)
    plsc.subcore_barrier()

    mvec = m_v[...]
    unit = jnp.where(lax.iota(jnp.int32, 16) == 0, 1.0, 0.0)

    def load_idx(i, srcb, dstb, etb):
        base = wid * EPW + i * CH
        pltpu.sync_copy(src_hbm.at[pl.ds(base, CH)], srcb)
        pltpu.sync_copy(dst_hbm.at[pl.ds(base, CH)], dstb)
        pltpu.sync_copy(et_hbm.at[pl.ds(base, CH)], etb)

    def attn_scale_scatter(srcb, dstb, etb, rowshb, gsem):
        def attn(r, _):
            o = pl.multiple_of(r * 16, 16)
            sv = plsc.load_gather(s_v, [srcb[pl.ds(o, 16)]])
            dv = plsc.load_gather(d_v, [dstb[pl.ds(o, 16)]])
            al = _lrelu(sv + dv + etb[pl.ds(o, 16)])
            ex_v[pl.ds(o, 16)] = jnp.exp(al - mvec)
            return 0

        lax.fori_loop(0, CH // 16, attn, 0)
        pltpu.make_async_copy(h_hbm.at[srcb], rowshb, gsem).wait()

        def scale(e, _):
            x = plsc.load_gather(ex_v, [jnp.full((16,), e, jnp.int32)])
            for k in range(dh // 16):
                rowshb[e, pl.ds(k * 16, 16)] = rowshb[e, pl.ds(k * 16, 16)] * x
            exrows_v[e, pl.ds(0, 16)] = unit * x
            return 0

        lax.fori_loop(0, CH, scale, 0)
        pltpu.sync_copy(rowshb, acc_sh.at[dstb], add=True)
        pltpu.sync_copy(exrows_v, den_sh.at[dstb], add=True)

    load_idx(0, src_a, dst_a, et_a)
    pltpu.async_copy(h_hbm.at[src_a], rowsh_a, gsem_a)

    def pair(g, _):
        c0 = g * 2
        load_idx(c0 + 1, src_b, dst_b, et_b)
        pltpu.async_copy(h_hbm.at[src_b], rowsh_b, gsem_b)
        attn_scale_scatter(src_a, dst_a, et_a, rowsh_a, gsem_a)
        load_idx(c0 + 2, src_a, dst_a, et_a)
        pltpu.async_copy(h_hbm.at[src_a], rowsh_a, gsem_a)
        attn_scale_scatter(src_b, dst_b, et_b, rowsh_b, gsem_b)
        return 0

    lax.fori_loop(0, (NCHUNK - 1) // 2, pair, 0)
    attn_scale_scatter(src_a, dst_a, et_a, rowsh_a, gsem_a)
    plsc.subcore_barrier()

    @pl.when(sid < ZTILES)
    def _():
        pltpu.sync_copy(acc_sh.at[pl.ds(sid * ZROWS, ZROWS)],
                        acc_hbm.at[cid, pl.ds(sid * ZROWS, ZROWS)])
        pltpu.sync_copy(den_sh.at[pl.ds(sid * ZROWS, ZROWS)],
                        den_hbm.at[cid, pl.ds(sid * ZROWS, ZROWS)])


def _make_sc_layer(dh):
    @functools.partial(
        pl.kernel,
        out_type=(
            jax.ShapeDtypeStruct((NC, N, dh), jnp.float32),
            jax.ShapeDtypeStruct((NC, N, 16), jnp.float32),
        ),
        mesh=_MESH,
        compiler_params=pltpu.CompilerParams(needs_layout_passes=False, use_tc_tiling_on_sc=False),
        scratch_types=[
            pltpu.VMEM((N,), jnp.float32),
            pltpu.VMEM((N,), jnp.float32),
            pltpu.VMEM((16,), jnp.float32),
            pltpu.VMEM((CH,), jnp.int32),
            pltpu.VMEM((CH,), jnp.int32),
            pltpu.VMEM((CH,), jnp.float32),
            pltpu.VMEM((CH, dh), jnp.float32),
            pltpu.SemaphoreType.DMA,
            pltpu.VMEM((CH,), jnp.int32),
            pltpu.VMEM((CH,), jnp.int32),
            pltpu.VMEM((CH,), jnp.float32),
            pltpu.VMEM((CH, dh), jnp.float32),
            pltpu.SemaphoreType.DMA,
            pltpu.VMEM((CH,), jnp.float32),
            pltpu.VMEM((CH, 16), jnp.float32),
            pltpu.VMEM_SHARED((N, dh), jnp.float32),
            pltpu.VMEM_SHARED((N, 16), jnp.float32),
        ],
    )
    def sc_layer(src_hbm, dst_hbm, et_hbm, s_hbm, d_hbm, m_hbm, h_hbm,
                 zacc_hbm, zden_hbm, acc_hbm, den_hbm, *scratch):
        _sc_layer_body(dh, src_hbm, dst_hbm, et_hbm, s_hbm, d_hbm, m_hbm,
                       h_hbm, zacc_hbm, zden_hbm, acc_hbm, den_hbm, *scratch)

    return sc_layer


_sc_layer32 = _make_sc_layer(32)
_sc_layer64 = _make_sc_layer(64)


def kernel(x, edge_index, edge_attr, W1, as1, ad1, We1, ae1, b1,
           W2, as2, ad2, We2, ae2, b2, W3, as3, ad3, We3, ae3, b3, W4, b4):
    src = edge_index[0]
    dst = edge_index[1]

    z4 = jnp.zeros((N, 4), jnp.float32)
    z16 = jnp.zeros((N, 16), jnp.float32)
    z32 = jnp.zeros((N, 32), jnp.float32)
    z64 = jnp.zeros((N, 64), jnp.float32)

    ea_fl = edge_attr.reshape(_EB, 512)
    et4_2d, e1_2d, e2_2d, e3_2d, etmax = _tc_et(ea_fl, We1, ae1, We2, ae2, We3, ae3)
    et4 = et4_2d.reshape(E, 4)
    ett = (e1_2d.reshape(E), e2_2d.reshape(E), e3_2d.reshape(E))
    naccD = _sc_pre(et4, dst, z4)
    h1, s1, d1, sdmax1, etl1, etl2, etl3, etlmax = _tc_l1(
        x, W1, as1, ad1, naccD[0], naccD[1])

    emax = jnp.maximum(etmax[0], etlmax[0])

    M1 = sdmax1[0, 0] + sdmax1[0, 1] + emax[0]
    m1 = jnp.full((16,), M1, jnp.float32)
    accD1 = _sc_layer32(src, dst, ett[0], s1, d1, m1, h1, z32, z16)

    h2, s2, d2, sdmax2 = _tc_lj(64, accD1, h1, s1, d1, etl1, M1,
                                b1, W2, as2, ad2)
    M2 = sdmax2[0, 0] + sdmax2[0, 1] + emax[1]
    m2 = jnp.full((16,), M2, jnp.float32)
    accD2 = _sc_layer64(src, dst, ett[1], s2, d2, m2, h2, z64, z16)

    h3, s3, d3, sdmax3 = _tc_lj(64, accD2, h2, s2, d2, etl2, M2,
                                b2, W3, as3, ad3)
    M3 = sdmax3[0, 0] + sdmax3[0, 1] + emax[2]
    m3 = jnp.full((16,), M3, jnp.float32)
    accD3 = _sc_layer64(src, dst, ett[2], s3, d3, m3, h3, z64, z16)

    return _tc_l4(accD3, h3, s3, d3, etl3, M3, b3, W4, b4)


# speedup vs baseline: 27.0855x; 1.1374x over previous
import functools
import jax
import jax.numpy as jnp
from jax import lax
from jax.experimental import pallas as pl
from jax.experimental.pallas import tpu as pltpu, tpu_sc as plsc

N = 10000
E = 320000
NEG = 0.2
NC = 2
NS = 16
NW = NC * NS
EPW = E // NW
ROWS_PW = N // NS
CH = 400
NCHUNK = EPW // CH


def _lrelu(v):
    return jnp.where(v > 0, v, NEG * v)


_BE = 16000


def _tc_et_body(ea_ref, we1_ref, ae1_ref, we2_ref, ae2_ref, we3_ref, ae3_ref,
                e1_ref, e2_ref, e3_ref, etmax_ref):
    i = pl.program_id(0)
    ves = []
    for we_ref, ae_ref in ((we1_ref, ae1_ref), (we2_ref, ae2_ref), (we3_ref, ae3_ref)):
        ves.append(we_ref[...] @ ae_ref[...])
    V = jnp.stack(ves, axis=1)
    et = ea_ref[...] @ V
    o = i * _BE
    e1_ref[pl.ds(o, _BE)] = et[:, 0]
    e2_ref[pl.ds(o, _BE)] = et[:, 1]
    e3_ref[pl.ds(o, _BE)] = et[:, 2]
    bmax = jnp.max(et, axis=0)[None, :]

    @pl.when(i == 0)
    def _():
        etmax_ref[...] = bmax

    @pl.when(i > 0)
    def _():
        etmax_ref[...] = jnp.maximum(etmax_ref[...], bmax)


def _tc_et(ea, We1, ae1, We2, ae2, We3, ae3):
    wspec = [pl.BlockSpec(memory_space=pltpu.VMEM)] * 6
    return pl.pallas_call(
        _tc_et_body,
        grid=(E // _BE,),
        out_shape=(
            jax.ShapeDtypeStruct((E,), jnp.float32),
            jax.ShapeDtypeStruct((E,), jnp.float32),
            jax.ShapeDtypeStruct((E,), jnp.float32),
            jax.ShapeDtypeStruct((1, 3), jnp.float32),
        ),
        in_specs=[pl.BlockSpec((_BE, 4), lambda i: (i, 0))] + wspec,
        out_specs=(
            pl.BlockSpec(memory_space=pltpu.VMEM),
            pl.BlockSpec(memory_space=pltpu.VMEM),
            pl.BlockSpec(memory_space=pltpu.VMEM),
            pl.BlockSpec((1, 3), lambda i: (0, 0)),
        ),
    )(ea, We1, ae1, We2, ae2, We3, ae3)


def _tc_l1_body(x_ref, w_ref, as_ref, ad_ref, n0_ref, n1_ref,
                h_ref, s_ref, d_ref, sdmax_ref, e1_ref, e2_ref, e3_ref,
                etlmax_ref):
    h = x_ref[...] @ w_ref[...]
    h_ref[...] = h
    s = h @ as_ref[...]
    d = h @ ad_ref[...]
    s_ref[...] = s
    d_ref[...] = d
    sdmax_ref[...] = jnp.stack([s.max(), d.max()])[None, :]
    nacc = n0_ref[...] + n1_ref[...]
    deg = jnp.clip(nacc[:, 3], 1.0, None)
    maxes = []
    for j, er in enumerate((e1_ref, e2_ref, e3_ref)):
        etl = nacc[:, j] / deg
        er[...] = etl
        maxes.append(etl.max())
    etlmax_ref[...] = jnp.stack(maxes)[None, :]


def _tc_l1(x, W1, as1, ad1, nacc0, nacc1):
    return pl.pallas_call(
        _tc_l1_body,
        out_shape=(
            jax.ShapeDtypeStruct((N, 32), jnp.float32),
            jax.ShapeDtypeStruct((N,), jnp.float32),
            jax.ShapeDtypeStruct((N,), jnp.float32),
            jax.ShapeDtypeStruct((1, 2), jnp.float32),
            jax.ShapeDtypeStruct((N,), jnp.float32),
            jax.ShapeDtypeStruct((N,), jnp.float32),
            jax.ShapeDtypeStruct((N,), jnp.float32),
            jax.ShapeDtypeStruct((1, 3), jnp.float32),
        ),
    )(x, W1, as1, ad1, nacc0, nacc1)


def _combine(a0_ref, a1_ref, n0_ref, n1_ref, h_ref, s_ref, d_ref, etl_ref,
             m_ref, b_ref):
    al = _lrelu(s_ref[...] + d_ref[...] + etl_ref[...])
    exl = jnp.exp(al - m_ref[0])[:, None]
    acc = a0_ref[...] + a1_ref[...]
    den = (n0_ref[...] + n1_ref[...])[:, 0:1]
    out = (acc + exl * h_ref[...]) / (den + exl) + b_ref[...][None, :]
    return jnp.maximum(out, 0.0)


def _tc_lj_body(a0_ref, a1_ref, n0_ref, n1_ref, h_ref, s_ref, d_ref, etl_ref,
                m_ref, b_ref, w_ref, as_ref, ad_ref,
                hn_ref, sn_ref, dn_ref, sdmax_ref):
    out = _combine(a0_ref, a1_ref, n0_ref, n1_ref, h_ref, s_ref, d_ref,
                   etl_ref, m_ref, b_ref)
    hn = out @ w_ref[...]
    hn_ref[...] = hn
    s = hn @ as_ref[...]
    d = hn @ ad_ref[...]
    sn_ref[...] = s
    dn_ref[...] = d
    sdmax_ref[...] = jnp.stack([s.max(), d.max()])[None, :]


def _tc_lj(dh, accD, h_prev, s, d, etl_col, M, b_prev, W, a_s, a_d):
    m2 = jnp.full((1,), M, jnp.float32)
    acc, den = accD
    return pl.pallas_call(
        _tc_lj_body,
        out_shape=(
            jax.ShapeDtypeStruct((N, dh), jnp.float32),
            jax.ShapeDtypeStruct((N,), jnp.float32),
            jax.ShapeDtypeStruct((N,), jnp.float32),
            jax.ShapeDtypeStruct((1, 2), jnp.float32),
        ),
        in_specs=[pl.BlockSpec(memory_space=pltpu.VMEM)] * 8
        + [pl.BlockSpec(memory_space=pltpu.SMEM)]
        + [pl.BlockSpec(memory_space=pltpu.VMEM)] * 4,
    )(acc[0], acc[1], den[0], den[1], h_prev, s, d, etl_col, m2, b_prev,
      W, a_s, a_d)


def _tc_l4_body(a0_ref, a1_ref, n0_ref, n1_ref, h_ref, s_ref, d_ref, etl_ref,
                m_ref, b3_ref, w4_ref, b4_ref, y_ref):
    out = _combine(a0_ref, a1_ref, n0_ref, n1_ref, h_ref, s_ref, d_ref,
                   etl_ref, m_ref, b3_ref)
    y_ref[...] = out @ w4_ref[...] + b4_ref[...][None, :]


def _tc_l4(accD, h3, s3, d3, etl3, M3, b3, W4, b4):
    m2 = jnp.full((1,), M3, jnp.float32)
    acc, den = accD
    return pl.pallas_call(
        _tc_l4_body,
        out_shape=jax.ShapeDtypeStruct((N, 512), jnp.float32),
        in_specs=[pl.BlockSpec(memory_space=pltpu.VMEM)] * 8
        + [pl.BlockSpec(memory_space=pltpu.SMEM)]
        + [pl.BlockSpec(memory_space=pltpu.VMEM)] * 3,
    )(acc[0], acc[1], den[0], den[1], h3, s3, d3, etl3, m2, b3, W4, b4)


_MESH = plsc.VectorSubcoreMesh(core_axis_name="c", subcore_axis_name="s")


ZROWS = 1000
ZTILES = N // ZROWS


def _zero_spmem_slice(acc_sh, zeros_hbm, sid):
    @pl.when(sid < ZTILES)
    def _():
        base = sid * ZROWS
        pltpu.sync_copy(zeros_hbm.at[pl.ds(base, ZROWS)],
                        acc_sh.at[pl.ds(base, ZROWS)])


def _sc_pre_body(e1_hbm, e2_hbm, e3_hbm, dst_hbm, z_hbm, out_hbm,
                 e1_v, e2_v, e3_v, dst_v, rows_v, acc_sh):
    cid = lax.axis_index("c")
    sid = lax.axis_index("s")
    wid = cid * NS + sid

    _zero_spmem_slice(acc_sh, z_hbm, sid)
    plsc.subcore_barrier()

    lane = lax.iota(jnp.int32, 16)
    u0 = jnp.where(lane == 0, 1.0, 0.0)
    u1 = jnp.where(lane == 1, 1.0, 0.0)
    u2 = jnp.where(lane == 2, 1.0, 0.0)
    u3 = jnp.where(lane == 3, 1.0, 0.0)

    def chunk(i, _):
        base = wid * EPW + i * CH
        pltpu.sync_copy(dst_hbm.at[pl.ds(base, CH)], dst_v)
        pltpu.sync_copy(e1_hbm.at[pl.ds(base, CH)], e1_v)
        pltpu.sync_copy(e2_hbm.at[pl.ds(base, CH)], e2_v)
        pltpu.sync_copy(e3_hbm.at[pl.ds(base, CH)], e3_v)

        def build(e, _):
            ix = jnp.full((16,), e, jnp.int32)
            x1 = plsc.load_gather(e1_v, [ix])
            x2 = plsc.load_gather(e2_v, [ix])
            x3 = plsc.load_gather(e3_v, [ix])
            rows_v[e, pl.ds(0, 16)] = x1 * u0 + x2 * u1 + (x3 * u2 + u3)
            return 0

        lax.fori_loop(0, CH, build, 0)
        pltpu.sync_copy(rows_v, acc_sh.at[dst_v], add=True)
        return 0

    lax.fori_loop(0, NCHUNK, chunk, 0)
    plsc.subcore_barrier()

    @pl.when(sid < ZTILES)
    def _():
        pltpu.sync_copy(acc_sh.at[pl.ds(sid * ZROWS, ZROWS)],
                        out_hbm.at[cid, pl.ds(sid * ZROWS, ZROWS)])


@functools.partial(
    pl.kernel,
    out_type=jax.ShapeDtypeStruct((NC, N, 16), jnp.float32),
    mesh=_MESH,
    compiler_params=pltpu.CompilerParams(needs_layout_passes=False, use_tc_tiling_on_sc=False),
    scratch_types=[
        pltpu.VMEM((CH,), jnp.float32),
        pltpu.VMEM((CH,), jnp.float32),
        pltpu.VMEM((CH,), jnp.float32),
        pltpu.VMEM((CH,), jnp.int32),
        pltpu.VMEM((CH, 16), jnp.float32),
        pltpu.VMEM_SHARED((N, 16), jnp.float32),
    ],
)
def _sc_pre(e1_hbm, e2_hbm, e3_hbm, dst_hbm, z_hbm, out_hbm, *scratch):
    _sc_pre_body(e1_hbm, e2_hbm, e3_hbm, dst_hbm, z_hbm, out_hbm, *scratch)


def _sc_layer_body(dh, src_hbm, dst_hbm, et_hbm, s_hbm, d_hbm, m_hbm, h_hbm,
                   zacc_hbm, zden_hbm, acc_hbm, den_hbm, s_v, d_v, m_v,
                   src_a, dst_a, et_a, rowsh_a, gsem_a,
                   src_b, dst_b, et_b, rowsh_b, gsem_b,
                   ex_v, exrows_v, acc_sh, den_sh):
    cid = lax.axis_index("c")
    sid = lax.axis_index("s")
    wid = cid * NS + sid

    pltpu.sync_copy(s_hbm, s_v)
    pltpu.sync_copy(d_hbm, d_v)
    pltpu.sync_copy(m_hbm, m_v)
    _zero_spmem_slice(acc_sh, zacc_hbm, sid)
    _zero_spmem_slice(den_sh, zden_hbm, sid)
    plsc.subcore_barrier()

    mvec = m_v[...]
    unit = jnp.where(lax.iota(jnp.int32, 16) == 0, 1.0, 0.0)

    def load_idx(i, srcb, dstb, etb):
        base = wid * EPW + i * CH
        pltpu.sync_copy(src_hbm.at[pl.ds(base, CH)], srcb)
        pltpu.sync_copy(dst_hbm.at[pl.ds(base, CH)], dstb)
        pltpu.sync_copy(et_hbm.at[pl.ds(base, CH)], etb)

    def attn_scale_scatter(srcb, dstb, etb, rowshb, gsem):
        def attn(r, _):
            o = pl.multiple_of(r * 16, 16)
            sv = plsc.load_gather(s_v, [srcb[pl.ds(o, 16)]])
            dv = plsc.load_gather(d_v, [dstb[pl.ds(o, 16)]])
            al = _lrelu(sv + dv + etb[pl.ds(o, 16)])
            ex_v[pl.ds(o, 16)] = jnp.exp(al - mvec)
            return 0

        lax.fori_loop(0, CH // 16, attn, 0)
        pltpu.make_async_copy(h_hbm.at[srcb], rowshb, gsem).wait()

        def scale(e, _):
            x = plsc.load_gather(ex_v, [jnp.full((16,), e, jnp.int32)])
            for k in range(dh // 16):
                rowshb[e, pl.ds(k * 16, 16)] = rowshb[e, pl.ds(k * 16, 16)] * x
            exrows_v[e, pl.ds(0, 16)] = unit * x
            return 0

        lax.fori_loop(0, CH, scale, 0)
        pltpu.sync_copy(rowshb, acc_sh.at[dstb], add=True)
        pltpu.sync_copy(exrows_v, den_sh.at[dstb], add=True)

    load_idx(0, src_a, dst_a, et_a)
    pltpu.async_copy(h_hbm.at[src_a], rowsh_a, gsem_a)

    def pair(g, _):
        c0 = g * 2
        load_idx(c0 + 1, src_b, dst_b, et_b)
        pltpu.async_copy(h_hbm.at[src_b], rowsh_b, gsem_b)
        attn_scale_scatter(src_a, dst_a, et_a, rowsh_a, gsem_a)
        load_idx(c0 + 2, src_a, dst_a, et_a)
        pltpu.async_copy(h_hbm.at[src_a], rowsh_a, gsem_a)
        attn_scale_scatter(src_b, dst_b, et_b, rowsh_b, gsem_b)
        return 0

    lax.fori_loop(0, (NCHUNK - 1) // 2, pair, 0)
    attn_scale_scatter(src_a, dst_a, et_a, rowsh_a, gsem_a)
    plsc.subcore_barrier()

    @pl.when(sid < ZTILES)
    def _():
        pltpu.sync_copy(acc_sh.at[pl.ds(sid * ZROWS, ZROWS)],
                        acc_hbm.at[cid, pl.ds(sid * ZROWS, ZROWS)])
        pltpu.sync_copy(den_sh.at[pl.ds(sid * ZROWS, ZROWS)],
                        den_hbm.at[cid, pl.ds(sid * ZROWS, ZROWS)])


def _make_sc_layer(dh):
    @functools.partial(
        pl.kernel,
        out_type=(
            jax.ShapeDtypeStruct((NC, N, dh), jnp.float32),
            jax.ShapeDtypeStruct((NC, N, 16), jnp.float32),
        ),
        mesh=_MESH,
        compiler_params=pltpu.CompilerParams(needs_layout_passes=False, use_tc_tiling_on_sc=False),
        scratch_types=[
            pltpu.VMEM((N,), jnp.float32),
            pltpu.VMEM((N,), jnp.float32),
            pltpu.VMEM((16,), jnp.float32),
            pltpu.VMEM((CH,), jnp.int32),
            pltpu.VMEM((CH,), jnp.int32),
            pltpu.VMEM((CH,), jnp.float32),
            pltpu.VMEM((CH, dh), jnp.float32),
            pltpu.SemaphoreType.DMA,
            pltpu.VMEM((CH,), jnp.int32),
            pltpu.VMEM((CH,), jnp.int32),
            pltpu.VMEM((CH,), jnp.float32),
            pltpu.VMEM((CH, dh), jnp.float32),
            pltpu.SemaphoreType.DMA,
            pltpu.VMEM((CH,), jnp.float32),
            pltpu.VMEM((CH, 16), jnp.float32),
            pltpu.VMEM_SHARED((N, dh), jnp.float32),
            pltpu.VMEM_SHARED((N, 16), jnp.float32),
        ],
    )
    def sc_layer(src_hbm, dst_hbm, et_hbm, s_hbm, d_hbm, m_hbm, h_hbm,
                 zacc_hbm, zden_hbm, acc_hbm, den_hbm, *scratch):
        _sc_layer_body(dh, src_hbm, dst_hbm, et_hbm, s_hbm, d_hbm, m_hbm,
                       h_hbm, zacc_hbm, zden_hbm, acc_hbm, den_hbm, *scratch)

    return sc_layer


_sc_layer32 = _make_sc_layer(32)
_sc_layer64 = _make_sc_layer(64)


def kernel(x, edge_index, edge_attr, W1, as1, ad1, We1, ae1, b1,
           W2, as2, ad2, We2, ae2, b2, W3, as3, ad3, We3, ae3, b3, W4, b4):
    src = edge_index[0]
    dst = edge_index[1]

    z16 = jnp.zeros((N, 16), jnp.float32)
    z32 = jnp.zeros((N, 32), jnp.float32)
    z64 = jnp.zeros((N, 64), jnp.float32)

    e1, e2, e3, etmax = _tc_et(edge_attr, We1, ae1, We2, ae2, We3, ae3)
    ett = (e1, e2, e3)
    naccD = _sc_pre(e1, e2, e3, dst, z16)
    h1, s1, d1, sdmax1, etl1, etl2, etl3, etlmax = _tc_l1(
        x, W1, as1, ad1, naccD[0], naccD[1])

    emax = jnp.maximum(etmax[0], etlmax[0])

    M1 = sdmax1[0, 0] + sdmax1[0, 1] + emax[0]
    m1 = jnp.full((16,), M1, jnp.float32)
    accD1 = _sc_layer32(src, dst, ett[0], s1, d1, m1, h1, z32, z16)

    h2, s2, d2, sdmax2 = _tc_lj(64, accD1, h1, s1, d1, etl1, M1,
                                b1, W2, as2, ad2)
    M2 = sdmax2[0, 0] + sdmax2[0, 1] + emax[1]
    m2 = jnp.full((16,), M2, jnp.float32)
    accD2 = _sc_layer64(src, dst, ett[1], s2, d2, m2, h2, z64, z16)

    h3, s3, d3, sdmax3 = _tc_lj(64, accD2, h2, s2, d2, etl2, M2,
                                b2, W3, as3, ad3)
    M3 = sdmax3[0, 0] + sdmax3[0, 1] + emax[2]
    m3 = jnp.full((16,), M3, jnp.float32)
    accD3 = _sc_layer64(src, dst, ett[2], s3, d3, m3, h3, z64, z16)

    return _tc_l4(accD3, h3, s3, d3, etl3, M3, b3, W4, b4)


# speedup vs baseline: 27.9742x; 1.0328x over previous
import functools
import jax
import jax.numpy as jnp
from jax import lax
from jax.experimental import pallas as pl
from jax.experimental.pallas import tpu as pltpu, tpu_sc as plsc

N = 10000
E = 320000
NEG = 0.2
NC = 2
NS = 16
NW = NC * NS
EPW = E // NW
ROWS_PW = N // NS
CH = 400
NCHUNK = EPW // CH


def _lrelu(v):
    return jnp.where(v > 0, v, NEG * v)


def _tc_a_body(x_ref, w_ref, as_ref, ad_ref, we1_ref, ae1_ref, we2_ref,
               ae2_ref, we3_ref, ae3_ref, ei_ref,
               h_ref, s_ref, d_ref, sdmax_ref, cs_ref, src_ref, dst_ref):
    h = x_ref[...] @ w_ref[...]
    h_ref[...] = h
    s = h @ as_ref[...]
    d = h @ ad_ref[...]
    s_ref[...] = s
    d_ref[...] = d
    sdmax_ref[...] = jnp.stack([s.max(), d.max()])[None, :]
    ves = []
    for we_r, ae_r in ((we1_ref, ae1_ref), (we2_ref, ae2_ref), (we3_ref, ae3_ref)):
        ves.append(we_r[...] @ ae_r[...])
    vflat = jnp.concatenate(ves)
    cs_ref[...] = vflat[:, None] * jnp.ones((1, 16), jnp.float32)
    ei = ei_ref[...]
    src_ref[...] = ei[0]
    dst_ref[...] = ei[1]


def _tc_a(x, W1, as1, ad1, We1, ae1, We2, ae2, We3, ae3, edge_index):
    return pl.pallas_call(
        _tc_a_body,
        out_shape=(
            jax.ShapeDtypeStruct((N, 32), jnp.float32),
            jax.ShapeDtypeStruct((N,), jnp.float32),
            jax.ShapeDtypeStruct((N,), jnp.float32),
            jax.ShapeDtypeStruct((1, 2), jnp.float32),
            jax.ShapeDtypeStruct((12, 16), jnp.float32),
            jax.ShapeDtypeStruct((E,), jnp.int32),
            jax.ShapeDtypeStruct((E,), jnp.int32),
        ),
    )(x, W1, as1, ad1, We1, ae1, We2, ae2, We3, ae3, edge_index)


def _tc_b_body(n0_ref, n1_ref, e1_ref, e2_ref, e3_ref, etlmax_ref):
    nacc = n0_ref[...] + n1_ref[...]
    deg = jnp.clip(nacc[:, 3], 1.0, None)
    maxes = []
    for j, er in enumerate((e1_ref, e2_ref, e3_ref)):
        etl = nacc[:, j] / deg
        er[...] = etl
        maxes.append(etl.max())
    etlmax_ref[...] = jnp.stack(maxes)[None, :]


def _tc_b(nacc0, nacc1):
    return pl.pallas_call(
        _tc_b_body,
        out_shape=(
            jax.ShapeDtypeStruct((N,), jnp.float32),
            jax.ShapeDtypeStruct((N,), jnp.float32),
            jax.ShapeDtypeStruct((N,), jnp.float32),
            jax.ShapeDtypeStruct((1, 3), jnp.float32),
        ),
    )(nacc0, nacc1)


def _combine(a0_ref, a1_ref, n0_ref, n1_ref, h_ref, s_ref, d_ref, etl_ref,
             m_ref, b_ref):
    al = _lrelu(s_ref[...] + d_ref[...] + etl_ref[...])
    exl = jnp.exp(al - m_ref[0])[:, None]
    acc = a0_ref[...] + a1_ref[...]
    den = (n0_ref[...] + n1_ref[...])[:, 0:1]
    out = (acc + exl * h_ref[...]) / (den + exl) + b_ref[...][None, :]
    return jnp.maximum(out, 0.0)


def _tc_lj_body(a0_ref, a1_ref, n0_ref, n1_ref, h_ref, s_ref, d_ref, etl_ref,
                m_ref, b_ref, w_ref, as_ref, ad_ref,
                hn_ref, sn_ref, dn_ref, sdmax_ref):
    out = _combine(a0_ref, a1_ref, n0_ref, n1_ref, h_ref, s_ref, d_ref,
                   etl_ref, m_ref, b_ref)
    hn = out @ w_ref[...]
    hn_ref[...] = hn
    s = hn @ as_ref[...]
    d = hn @ ad_ref[...]
    sn_ref[...] = s
    dn_ref[...] = d
    sdmax_ref[...] = jnp.stack([s.max(), d.max()])[None, :]


def _tc_lj(dh, accD, h_prev, s, d, etl_col, M, b_prev, W, a_s, a_d):
    m2 = jnp.full((1,), M, jnp.float32)
    acc, den = accD
    return pl.pallas_call(
        _tc_lj_body,
        out_shape=(
            jax.ShapeDtypeStruct((N, dh), jnp.float32),
            jax.ShapeDtypeStruct((N,), jnp.float32),
            jax.ShapeDtypeStruct((N,), jnp.float32),
            jax.ShapeDtypeStruct((1, 2), jnp.float32),
        ),
        in_specs=[pl.BlockSpec(memory_space=pltpu.VMEM)] * 8
        + [pl.BlockSpec(memory_space=pltpu.SMEM)]
        + [pl.BlockSpec(memory_space=pltpu.VMEM)] * 4,
    )(acc[0], acc[1], den[0], den[1], h_prev, s, d, etl_col, m2, b_prev,
      W, a_s, a_d)


def _tc_l4_body(a0_ref, a1_ref, n0_ref, n1_ref, h_ref, s_ref, d_ref, etl_ref,
                m_ref, b3_ref, w4_ref, b4_ref, y_ref):
    out = _combine(a0_ref, a1_ref, n0_ref, n1_ref, h_ref, s_ref, d_ref,
                   etl_ref, m_ref, b3_ref)
    y_ref[...] = out @ w4_ref[...] + b4_ref[...][None, :]


def _tc_l4(accD, h3, s3, d3, etl3, M3, b3, W4, b4):
    m2 = jnp.full((1,), M3, jnp.float32)
    acc, den = accD
    return pl.pallas_call(
        _tc_l4_body,
        out_shape=jax.ShapeDtypeStruct((N, 512), jnp.float32),
        in_specs=[pl.BlockSpec(memory_space=pltpu.VMEM)] * 8
        + [pl.BlockSpec(memory_space=pltpu.SMEM)]
        + [pl.BlockSpec(memory_space=pltpu.VMEM)] * 3,
    )(acc[0], acc[1], den[0], den[1], h3, s3, d3, etl3, m2, b3, W4, b4)


_MESH = plsc.VectorSubcoreMesh(core_axis_name="c", subcore_axis_name="s")


ZROWS = 1000
ZTILES = N // ZROWS


def _zero_spmem_slice(acc_sh, zeros_hbm, sid):
    @pl.when(sid < ZTILES)
    def _():
        base = sid * ZROWS
        pltpu.sync_copy(zeros_hbm.at[pl.ds(base, ZROWS)],
                        acc_sh.at[pl.ds(base, ZROWS)])


def _sc_pre_body(ea_hbm, cs_hbm, dst_hbm, z_hbm,
                 out_hbm, et1_hbm, et2_hbm, et3_hbm, etmax_hbm,
                 ea_v, cs_v, dst_v, e1_v, e2_v, e3_v, rows_v, mrow_v, acc_sh):
    cid = lax.axis_index("c")
    sid = lax.axis_index("s")
    wid = cid * NS + sid

    pltpu.sync_copy(cs_hbm, cs_v)
    _zero_spmem_slice(acc_sh, z_hbm, sid)
    plsc.subcore_barrier()

    cs = [cs_v[r, pl.ds(0, 16)] for r in range(12)]
    lane = lax.iota(jnp.int32, 16)
    u0 = jnp.where(lane == 0, 1.0, 0.0)
    u1 = jnp.where(lane == 1, 1.0, 0.0)
    u2 = jnp.where(lane == 2, 1.0, 0.0)
    u3 = jnp.where(lane == 3, 1.0, 0.0)
    neg = jnp.full((16,), -3.0e38, jnp.float32)

    def chunk(i, carry):
        m1, m2, m3 = carry
        base = wid * EPW + i * CH
        pltpu.sync_copy(dst_hbm.at[pl.ds(base, CH)], dst_v)
        pltpu.sync_copy(ea_hbm.at[pl.ds(base, CH)], ea_v)

        def group(g, c):
            g1, g2, g3 = c
            o = pl.multiple_of(g * 16, 16)
            ridx = lane + o
            x = [plsc.load_gather(ea_v, [ridx, jnp.full((16,), k, jnp.int32)])
                 for k in range(4)]
            et1 = x[0] * cs[0] + x[1] * cs[1] + (x[2] * cs[2] + x[3] * cs[3])
            et2 = x[0] * cs[4] + x[1] * cs[5] + (x[2] * cs[6] + x[3] * cs[7])
            et3 = x[0] * cs[8] + x[1] * cs[9] + (x[2] * cs[10] + x[3] * cs[11])
            e1_v[pl.ds(o, 16)] = et1
            e2_v[pl.ds(o, 16)] = et2
            e3_v[pl.ds(o, 16)] = et3
            return (jnp.maximum(g1, et1), jnp.maximum(g2, et2),
                    jnp.maximum(g3, et3))

        m1, m2, m3 = lax.fori_loop(0, CH // 16, group, (m1, m2, m3))

        def build(e, _):
            ix = jnp.full((16,), e, jnp.int32)
            x1 = plsc.load_gather(e1_v, [ix])
            x2 = plsc.load_gather(e2_v, [ix])
            x3 = plsc.load_gather(e3_v, [ix])
            rows_v[e, pl.ds(0, 16)] = x1 * u0 + x2 * u1 + (x3 * u2 + u3)
            return 0

        lax.fori_loop(0, CH, build, 0)
        pltpu.sync_copy(rows_v, acc_sh.at[dst_v], add=True)
        pltpu.sync_copy(e1_v, et1_hbm.at[pl.ds(base, CH)])
        pltpu.sync_copy(e2_v, et2_hbm.at[pl.ds(base, CH)])
        pltpu.sync_copy(e3_v, et3_hbm.at[pl.ds(base, CH)])
        return (m1, m2, m3)

    m1, m2, m3 = lax.fori_loop(0, NCHUNK, chunk, (neg, neg, neg))
    mrow_v[pl.ds(0, 16)] = m1
    mrow_v[pl.ds(16, 16)] = m2
    mrow_v[pl.ds(32, 16)] = m3
    pltpu.sync_copy(mrow_v, etmax_hbm.at[cid, sid])
    plsc.subcore_barrier()

    @pl.when(sid < ZTILES)
    def _():
        pltpu.sync_copy(acc_sh.at[pl.ds(sid * ZROWS, ZROWS)],
                        out_hbm.at[cid, pl.ds(sid * ZROWS, ZROWS)])


@functools.partial(
    pl.kernel,
    out_type=(
        jax.ShapeDtypeStruct((NC, N, 16), jnp.float32),
        jax.ShapeDtypeStruct((E,), jnp.float32),
        jax.ShapeDtypeStruct((E,), jnp.float32),
        jax.ShapeDtypeStruct((E,), jnp.float32),
        jax.ShapeDtypeStruct((NC, NS, 48), jnp.float32),
    ),
    mesh=_MESH,
    compiler_params=pltpu.CompilerParams(needs_layout_passes=False, use_tc_tiling_on_sc=False),
    scratch_types=[
        pltpu.VMEM((CH, 4), jnp.float32),
        pltpu.VMEM((12, 16), jnp.float32),
        pltpu.VMEM((CH,), jnp.int32),
        pltpu.VMEM((CH,), jnp.float32),
        pltpu.VMEM((CH,), jnp.float32),
        pltpu.VMEM((CH,), jnp.float32),
        pltpu.VMEM((CH, 16), jnp.float32),
        pltpu.VMEM((48,), jnp.float32),
        pltpu.VMEM_SHARED((N, 16), jnp.float32),
    ],
)
def _sc_pre(ea_hbm, cs_hbm, dst_hbm, z_hbm, out_hbm, et1_hbm, et2_hbm,
            et3_hbm, etmax_hbm, *scratch):
    _sc_pre_body(ea_hbm, cs_hbm, dst_hbm, z_hbm, out_hbm, et1_hbm, et2_hbm,
                 et3_hbm, etmax_hbm, *scratch)


def _sc_layer_body(dh, src_hbm, dst_hbm, et_hbm, s_hbm, d_hbm, m_hbm, h_hbm,
                   zacc_hbm, zden_hbm, acc_hbm, den_hbm, s_v, d_v, m_v,
                   src_a, dst_a, et_a, rowsh_a, gsem_a,
                   src_b, dst_b, et_b, rowsh_b, gsem_b,
                   ex_v, exrows_v, acc_sh, den_sh):
    cid = lax.axis_index("c")
    sid = lax.axis_index("s")
    wid = cid * NS + sid

    pltpu.sync_copy(s_hbm, s_v)
    pltpu.sync_copy(d_hbm, d_v)
    pltpu.sync_copy(m_hbm, m_v)
    _zero_spmem_slice(acc_sh, zacc_hbm, sid)
    _zero_spmem_slice(den_sh, zden_hbm, sid)
    plsc.subcore_barrier()

    mvec = m_v[...]
    unit = jnp.where(lax.iota(jnp.int32, 16) == 0, 1.0, 0.0)

    def load_idx(i, srcb, dstb, etb):
        base = wid * EPW + i * CH
        pltpu.sync_copy(src_hbm.at[pl.ds(base, CH)], srcb)
        pltpu.sync_copy(dst_hbm.at[pl.ds(base, CH)], dstb)
        pltpu.sync_copy(et_hbm.at[pl.ds(base, CH)], etb)

    def attn_scale_scatter(srcb, dstb, etb, rowshb, gsem):
        def attn(r, _):
            o = pl.multiple_of(r * 16, 16)
            sv = plsc.load_gather(s_v, [srcb[pl.ds(o, 16)]])
            dv = plsc.load_gather(d_v, [dstb[pl.ds(o, 16)]])
            al = _lrelu(sv + dv + etb[pl.ds(o, 16)])
            ex_v[pl.ds(o, 16)] = jnp.exp(al - mvec)
            return 0

        lax.fori_loop(0, CH // 16, attn, 0)
        pltpu.make_async_copy(h_hbm.at[srcb], rowshb, gsem).wait()

        def scale(e, _):
            x = plsc.load_gather(ex_v, [jnp.full((16,), e, jnp.int32)])
            for k in range(dh // 16):
                rowshb[e, pl.ds(k * 16, 16)] = rowshb[e, pl.ds(k * 16, 16)] * x
            exrows_v[e, pl.ds(0, 16)] = unit * x
            return 0

        lax.fori_loop(0, CH, scale, 0)
        pltpu.sync_copy(rowshb, acc_sh.at[dstb], add=True)
        pltpu.sync_copy(exrows_v, den_sh.at[dstb], add=True)

    load_idx(0, src_a, dst_a, et_a)
    pltpu.async_copy(h_hbm.at[src_a], rowsh_a, gsem_a)

    def pair(g, _):
        c0 = g * 2
        load_idx(c0 + 1, src_b, dst_b, et_b)
        pltpu.async_copy(h_hbm.at[src_b], rowsh_b, gsem_b)
        attn_scale_scatter(src_a, dst_a, et_a, rowsh_a, gsem_a)
        load_idx(c0 + 2, src_a, dst_a, et_a)
        pltpu.async_copy(h_hbm.at[src_a], rowsh_a, gsem_a)
        attn_scale_scatter(src_b, dst_b, et_b, rowsh_b, gsem_b)
        return 0

    lax.fori_loop(0, (NCHUNK - 1) // 2, pair, 0)
    attn_scale_scatter(src_a, dst_a, et_a, rowsh_a, gsem_a)
    plsc.subcore_barrier()

    @pl.when(sid < ZTILES)
    def _():
        pltpu.sync_copy(acc_sh.at[pl.ds(sid * ZROWS, ZROWS)],
                        acc_hbm.at[cid, pl.ds(sid * ZROWS, ZROWS)])
        pltpu.sync_copy(den_sh.at[pl.ds(sid * ZROWS, ZROWS)],
                        den_hbm.at[cid, pl.ds(sid * ZROWS, ZROWS)])


def _make_sc_layer(dh):
    @functools.partial(
        pl.kernel,
        out_type=(
            jax.ShapeDtypeStruct((NC, N, dh), jnp.float32),
            jax.ShapeDtypeStruct((NC, N, 16), jnp.float32),
        ),
        mesh=_MESH,
        compiler_params=pltpu.CompilerParams(needs_layout_passes=False, use_tc_tiling_on_sc=False),
        scratch_types=[
            pltpu.VMEM((N,), jnp.float32),
            pltpu.VMEM((N,), jnp.float32),
            pltpu.VMEM((16,), jnp.float32),
            pltpu.VMEM((CH,), jnp.int32),
            pltpu.VMEM((CH,), jnp.int32),
            pltpu.VMEM((CH,), jnp.float32),
            pltpu.VMEM((CH, dh), jnp.float32),
            pltpu.SemaphoreType.DMA,
            pltpu.VMEM((CH,), jnp.int32),
            pltpu.VMEM((CH,), jnp.int32),
            pltpu.VMEM((CH,), jnp.float32),
            pltpu.VMEM((CH, dh), jnp.float32),
            pltpu.SemaphoreType.DMA,
            pltpu.VMEM((CH,), jnp.float32),
            pltpu.VMEM((CH, 16), jnp.float32),
            pltpu.VMEM_SHARED((N, dh), jnp.float32),
            pltpu.VMEM_SHARED((N, 16), jnp.float32),
        ],
    )
    def sc_layer(src_hbm, dst_hbm, et_hbm, s_hbm, d_hbm, m_hbm, h_hbm,
                 zacc_hbm, zden_hbm, acc_hbm, den_hbm, *scratch):
        _sc_layer_body(dh, src_hbm, dst_hbm, et_hbm, s_hbm, d_hbm, m_hbm,
                       h_hbm, zacc_hbm, zden_hbm, acc_hbm, den_hbm, *scratch)

    return sc_layer


_sc_layer32 = _make_sc_layer(32)
_sc_layer64 = _make_sc_layer(64)


def kernel(x, edge_index, edge_attr, W1, as1, ad1, We1, ae1, b1,
           W2, as2, ad2, We2, ae2, b2, W3, as3, ad3, We3, ae3, b3, W4, b4):
    z16 = jnp.zeros((N, 16), jnp.float32)
    z32 = jnp.zeros((N, 32), jnp.float32)
    z64 = jnp.zeros((N, 64), jnp.float32)

    h1, s1, d1, sdmax1, CS, src, dst = _tc_a(
        x, W1, as1, ad1, We1, ae1, We2, ae2, We3, ae3, edge_index)
    naccD, e1, e2, e3, etmaxT = _sc_pre(edge_attr, CS, dst, z16)
    ett = (e1, e2, e3)
    etl1, etl2, etl3, etlmax = _tc_b(naccD[0], naccD[1])

    etmax = jnp.max(etmaxT.reshape(NC, NS, 3, 16), axis=(0, 1, 3))
    emax = jnp.maximum(etmax, etlmax[0])

    M1 = sdmax1[0, 0] + sdmax1[0, 1] + emax[0]
    m1 = jnp.full((16,), M1, jnp.float32)
    accD1 = _sc_layer32(src, dst, ett[0], s1, d1, m1, h1, z32, z16)

    h2, s2, d2, sdmax2 = _tc_lj(64, accD1, h1, s1, d1, etl1, M1,
                                b1, W2, as2, ad2)
    M2 = sdmax2[0, 0] + sdmax2[0, 1] + emax[1]
    m2 = jnp.full((16,), M2, jnp.float32)
    accD2 = _sc_layer64(src, dst, ett[1], s2, d2, m2, h2, z64, z16)

    h3, s3, d3, sdmax3 = _tc_lj(64, accD2, h2, s2, d2, etl2, M2,
                                b2, W3, as3, ad3)
    M3 = sdmax3[0, 0] + sdmax3[0, 1] + emax[2]
    m3 = jnp.full((16,), M3, jnp.float32)
    accD3 = _sc_layer64(src, dst, ett[2], s3, d3, m3, h3, z64, z16)

    return _tc_l4(accD3, h3, s3, d3, etl3, M3, b3, W4, b4)


# speedup vs baseline: 30.0441x; 1.0740x over previous
import functools
import jax
import jax.numpy as jnp
from jax import lax
from jax.experimental import pallas as pl
from jax.experimental.pallas import tpu as pltpu, tpu_sc as plsc

N = 10000
E = 320000
NEG = 0.2
NC = 2
NS = 16
NW = NC * NS
EPW = E // NW
ROWS_PW = N // NS
CH = 400
NCHUNK = EPW // CH


def _lrelu(v):
    return jnp.where(v > 0, v, NEG * v)


def _tc_a_body(x_ref, w_ref, as_ref, ad_ref, we1_ref, ae1_ref, we2_ref,
               ae2_ref, we3_ref, ae3_ref, ei_ref,
               h_ref, s_ref, d_ref, sdmax_ref, cs_ref, src_ref, dst_ref):
    h = x_ref[...] @ w_ref[...]
    h_ref[...] = h
    s = h @ as_ref[...]
    d = h @ ad_ref[...]
    s_ref[...] = s
    d_ref[...] = d
    sdmax_ref[...] = jnp.stack([s.max(), d.max()])[None, :]
    ves = []
    for we_r, ae_r in ((we1_ref, ae1_ref), (we2_ref, ae2_ref), (we3_ref, ae3_ref)):
        ves.append(we_r[...] @ ae_r[...])
    vflat = jnp.concatenate(ves)
    cs_ref[...] = vflat[:, None] * jnp.ones((1, 16), jnp.float32)
    ei = ei_ref[...]
    src_ref[...] = ei[0]
    dst_ref[...] = ei[1]


def _tc_a(x, W1, as1, ad1, We1, ae1, We2, ae2, We3, ae3, edge_index):
    return pl.pallas_call(
        _tc_a_body,
        out_shape=(
            jax.ShapeDtypeStruct((N, 32), jnp.float32),
            jax.ShapeDtypeStruct((N,), jnp.float32),
            jax.ShapeDtypeStruct((N,), jnp.float32),
            jax.ShapeDtypeStruct((1, 2), jnp.float32),
            jax.ShapeDtypeStruct((12, 16), jnp.float32),
            jax.ShapeDtypeStruct((E,), jnp.int32),
            jax.ShapeDtypeStruct((E,), jnp.int32),
        ),
    )(x, W1, as1, ad1, We1, ae1, We2, ae2, We3, ae3, edge_index)


def _tc_b_body(n0_ref, n1_ref, e1_ref, e2_ref, e3_ref, etlmax_ref):
    nacc = n0_ref[...] + n1_ref[...]
    deg = jnp.clip(nacc[:, 3], 1.0, None)
    maxes = []
    for j, er in enumerate((e1_ref, e2_ref, e3_ref)):
        etl = nacc[:, j] / deg
        er[...] = etl
        maxes.append(etl.max())
    etlmax_ref[...] = jnp.stack(maxes)[None, :]


def _tc_b(nacc0, nacc1):
    return pl.pallas_call(
        _tc_b_body,
        out_shape=(
            jax.ShapeDtypeStruct((N,), jnp.float32),
            jax.ShapeDtypeStruct((N,), jnp.float32),
            jax.ShapeDtypeStruct((N,), jnp.float32),
            jax.ShapeDtypeStruct((1, 3), jnp.float32),
        ),
    )(nacc0, nacc1)


def _combine(a0_ref, a1_ref, n0_ref, n1_ref, h_ref, s_ref, d_ref, etl_ref,
             m_ref, b_ref):
    al = _lrelu(s_ref[...] + d_ref[...] + etl_ref[...])
    exl = jnp.exp(al - m_ref[0])[:, None]
    acc = a0_ref[...] + a1_ref[...]
    den = (n0_ref[...] + n1_ref[...])[:, 0:1]
    out = (acc + exl * h_ref[...]) / (den + exl) + b_ref[...][None, :]
    return jnp.maximum(out, 0.0)


def _tc_lj_body(a0_ref, a1_ref, n0_ref, n1_ref, h_ref, s_ref, d_ref, etl_ref,
                m_ref, b_ref, w_ref, as_ref, ad_ref,
                hn_ref, sn_ref, dn_ref, sdmax_ref):
    out = _combine(a0_ref, a1_ref, n0_ref, n1_ref, h_ref, s_ref, d_ref,
                   etl_ref, m_ref, b_ref)
    hn = out @ w_ref[...]
    hn_ref[...] = hn
    s = hn @ as_ref[...]
    d = hn @ ad_ref[...]
    sn_ref[...] = s
    dn_ref[...] = d
    sdmax_ref[...] = jnp.stack([s.max(), d.max()])[None, :]


def _tc_lj(dh, accD, h_prev, s, d, etl_col, M, b_prev, W, a_s, a_d):
    m2 = jnp.full((1,), M, jnp.float32)
    acc, den = accD
    return pl.pallas_call(
        _tc_lj_body,
        out_shape=(
            jax.ShapeDtypeStruct((N, dh), jnp.float32),
            jax.ShapeDtypeStruct((N,), jnp.float32),
            jax.ShapeDtypeStruct((N,), jnp.float32),
            jax.ShapeDtypeStruct((1, 2), jnp.float32),
        ),
        in_specs=[pl.BlockSpec(memory_space=pltpu.VMEM)] * 8
        + [pl.BlockSpec(memory_space=pltpu.SMEM)]
        + [pl.BlockSpec(memory_space=pltpu.VMEM)] * 4,
    )(acc[0], acc[1], den[0], den[1], h_prev, s, d, etl_col, m2, b_prev,
      W, a_s, a_d)


def _tc_l4_body(a0_ref, a1_ref, n0_ref, n1_ref, h_ref, s_ref, d_ref, etl_ref,
                m_ref, b3_ref, w4_ref, b4_ref, y_ref):
    out = _combine(a0_ref, a1_ref, n0_ref, n1_ref, h_ref, s_ref, d_ref,
                   etl_ref, m_ref, b3_ref)
    y_ref[...] = out @ w4_ref[...] + b4_ref[...][None, :]


def _tc_l4(accD, h3, s3, d3, etl3, M3, b3, W4, b4):
    m2 = jnp.full((1,), M3, jnp.float32)
    acc, den = accD
    return pl.pallas_call(
        _tc_l4_body,
        out_shape=jax.ShapeDtypeStruct((N, 512), jnp.float32),
        in_specs=[pl.BlockSpec(memory_space=pltpu.VMEM)] * 8
        + [pl.BlockSpec(memory_space=pltpu.SMEM)]
        + [pl.BlockSpec(memory_space=pltpu.VMEM)] * 3,
    )(acc[0], acc[1], den[0], den[1], h3, s3, d3, etl3, m2, b3, W4, b4)


_MESH = plsc.VectorSubcoreMesh(core_axis_name="c", subcore_axis_name="s")


ZROWS = 1000
ZTILES = N // ZROWS


def _zero_spmem_slice(acc_sh, zeros_hbm, sid):
    @pl.when(sid < ZTILES)
    def _():
        base = sid * ZROWS
        pltpu.sync_copy(zeros_hbm.at[pl.ds(base, ZROWS)],
                        acc_sh.at[pl.ds(base, ZROWS)])


def _sc_pre_body(ea_hbm, cs_hbm, dst_hbm, z_hbm,
                 out_hbm, et1_hbm, et2_hbm, et3_hbm, etmax_hbm,
                 ea_v, cs_v, dst_v, e1_v, e2_v, e3_v, rows_v, mrow_v, acc_sh):
    cid = lax.axis_index("c")
    sid = lax.axis_index("s")
    wid = cid * NS + sid

    pltpu.sync_copy(cs_hbm, cs_v)
    _zero_spmem_slice(acc_sh, z_hbm, sid)
    plsc.subcore_barrier()

    cs = [cs_v[r, pl.ds(0, 16)] for r in range(12)]
    lane = lax.iota(jnp.int32, 16)
    u0 = jnp.where(lane == 0, 1.0, 0.0)
    u1 = jnp.where(lane == 1, 1.0, 0.0)
    u2 = jnp.where(lane == 2, 1.0, 0.0)
    u3 = jnp.where(lane == 3, 1.0, 0.0)
    neg = jnp.full((16,), -3.0e38, jnp.float32)

    def chunk(i, carry):
        m1, m2, m3 = carry
        base = wid * EPW + i * CH
        pltpu.sync_copy(dst_hbm.at[pl.ds(base, CH)], dst_v)
        pltpu.sync_copy(ea_hbm.at[pl.ds(base * 4, CH * 4)], ea_v)

        def group(g, c):
            g1, g2, g3 = c
            o = pl.multiple_of(g * 16, 16)
            ridx = (lane + o) * 4
            x = [plsc.load_gather(ea_v, [ridx + k]) for k in range(4)]
            et1 = x[0] * cs[0] + x[1] * cs[1] + (x[2] * cs[2] + x[3] * cs[3])
            et2 = x[0] * cs[4] + x[1] * cs[5] + (x[2] * cs[6] + x[3] * cs[7])
            et3 = x[0] * cs[8] + x[1] * cs[9] + (x[2] * cs[10] + x[3] * cs[11])
            e1_v[pl.ds(o, 16)] = et1
            e2_v[pl.ds(o, 16)] = et2
            e3_v[pl.ds(o, 16)] = et3
            return (jnp.maximum(g1, et1), jnp.maximum(g2, et2),
                    jnp.maximum(g3, et3))

        m1, m2, m3 = lax.fori_loop(0, CH // 16, group, (m1, m2, m3))

        def build(e, _):
            ix = jnp.full((16,), e, jnp.int32)
            x1 = plsc.load_gather(e1_v, [ix])
            x2 = plsc.load_gather(e2_v, [ix])
            x3 = plsc.load_gather(e3_v, [ix])
            rows_v[e, pl.ds(0, 16)] = x1 * u0 + x2 * u1 + (x3 * u2 + u3)
            return 0

        lax.fori_loop(0, CH, build, 0)
        pltpu.sync_copy(rows_v, acc_sh.at[dst_v], add=True)
        pltpu.sync_copy(e1_v, et1_hbm.at[pl.ds(base, CH)])
        pltpu.sync_copy(e2_v, et2_hbm.at[pl.ds(base, CH)])
        pltpu.sync_copy(e3_v, et3_hbm.at[pl.ds(base, CH)])
        return (m1, m2, m3)

    m1, m2, m3 = lax.fori_loop(0, NCHUNK, chunk, (neg, neg, neg))
    mrow_v[pl.ds(0, 16)] = m1
    mrow_v[pl.ds(16, 16)] = m2
    mrow_v[pl.ds(32, 16)] = m3
    pltpu.sync_copy(mrow_v, etmax_hbm.at[cid, sid])
    plsc.subcore_barrier()

    @pl.when(sid < ZTILES)
    def _():
        pltpu.sync_copy(acc_sh.at[pl.ds(sid * ZROWS, ZROWS)],
                        out_hbm.at[cid, pl.ds(sid * ZROWS, ZROWS)])


@functools.partial(
    pl.kernel,
    out_type=(
        jax.ShapeDtypeStruct((NC, N, 16), jnp.float32),
        jax.ShapeDtypeStruct((E,), jnp.float32),
        jax.ShapeDtypeStruct((E,), jnp.float32),
        jax.ShapeDtypeStruct((E,), jnp.float32),
        jax.ShapeDtypeStruct((NC, NS, 48), jnp.float32),
    ),
    mesh=_MESH,
    compiler_params=pltpu.CompilerParams(needs_layout_passes=False, use_tc_tiling_on_sc=False),
    scratch_types=[
        pltpu.VMEM((CH * 4,), jnp.float32),
        pltpu.VMEM((12, 16), jnp.float32),
        pltpu.VMEM((CH,), jnp.int32),
        pltpu.VMEM((CH,), jnp.float32),
        pltpu.VMEM((CH,), jnp.float32),
        pltpu.VMEM((CH,), jnp.float32),
        pltpu.VMEM((CH, 16), jnp.float32),
        pltpu.VMEM((48,), jnp.float32),
        pltpu.VMEM_SHARED((N, 16), jnp.float32),
    ],
)
def _sc_pre(ea_hbm, cs_hbm, dst_hbm, z_hbm, out_hbm, et1_hbm, et2_hbm,
            et3_hbm, etmax_hbm, *scratch):
    _sc_pre_body(ea_hbm, cs_hbm, dst_hbm, z_hbm, out_hbm, et1_hbm, et2_hbm,
                 et3_hbm, etmax_hbm, *scratch)


def _sc_layer_body(dh, src_hbm, dst_hbm, et_hbm, s_hbm, d_hbm, m_hbm, h_hbm,
                   zacc_hbm, zden_hbm, acc_hbm, den_hbm, s_v, d_v, m_v,
                   src_a, dst_a, et_a, rowsh_a, gsem_a,
                   src_b, dst_b, et_b, rowsh_b, gsem_b,
                   ex_v, exrows_v, acc_sh, den_sh):
    cid = lax.axis_index("c")
    sid = lax.axis_index("s")
    wid = cid * NS + sid

    pltpu.sync_copy(s_hbm, s_v)
    pltpu.sync_copy(d_hbm, d_v)
    pltpu.sync_copy(m_hbm, m_v)
    _zero_spmem_slice(acc_sh, zacc_hbm, sid)
    _zero_spmem_slice(den_sh, zden_hbm, sid)
    plsc.subcore_barrier()

    mvec = m_v[...]
    unit = jnp.where(lax.iota(jnp.int32, 16) == 0, 1.0, 0.0)

    def load_idx(i, srcb, dstb, etb):
        base = wid * EPW + i * CH
        pltpu.sync_copy(src_hbm.at[pl.ds(base, CH)], srcb)
        pltpu.sync_copy(dst_hbm.at[pl.ds(base, CH)], dstb)
        pltpu.sync_copy(et_hbm.at[pl.ds(base, CH)], etb)

    def attn_scale_scatter(srcb, dstb, etb, rowshb, gsem):
        def attn(r, _):
            o = pl.multiple_of(r * 16, 16)
            sv = plsc.load_gather(s_v, [srcb[pl.ds(o, 16)]])
            dv = plsc.load_gather(d_v, [dstb[pl.ds(o, 16)]])
            al = _lrelu(sv + dv + etb[pl.ds(o, 16)])
            ex_v[pl.ds(o, 16)] = jnp.exp(al - mvec)
            return 0

        lax.fori_loop(0, CH // 16, attn, 0)
        pltpu.make_async_copy(h_hbm.at[srcb], rowshb, gsem).wait()

        def scale(e, _):
            x = plsc.load_gather(ex_v, [jnp.full((16,), e, jnp.int32)])
            for k in range(dh // 16):
                rowshb[e, pl.ds(k * 16, 16)] = rowshb[e, pl.ds(k * 16, 16)] * x
            exrows_v[e, pl.ds(0, 16)] = unit * x
            return 0

        lax.fori_loop(0, CH, scale, 0)
        pltpu.sync_copy(rowshb, acc_sh.at[dstb], add=True)
        pltpu.sync_copy(exrows_v, den_sh.at[dstb], add=True)

    load_idx(0, src_a, dst_a, et_a)
    pltpu.async_copy(h_hbm.at[src_a], rowsh_a, gsem_a)

    def pair(g, _):
        c0 = g * 2
        load_idx(c0 + 1, src_b, dst_b, et_b)
        pltpu.async_copy(h_hbm.at[src_b], rowsh_b, gsem_b)
        attn_scale_scatter(src_a, dst_a, et_a, rowsh_a, gsem_a)
        load_idx(c0 + 2, src_a, dst_a, et_a)
        pltpu.async_copy(h_hbm.at[src_a], rowsh_a, gsem_a)
        attn_scale_scatter(src_b, dst_b, et_b, rowsh_b, gsem_b)
        return 0

    lax.fori_loop(0, (NCHUNK - 1) // 2, pair, 0)
    attn_scale_scatter(src_a, dst_a, et_a, rowsh_a, gsem_a)
    plsc.subcore_barrier()

    @pl.when(sid < ZTILES)
    def _():
        pltpu.sync_copy(acc_sh.at[pl.ds(sid * ZROWS, ZROWS)],
                        acc_hbm.at[cid, pl.ds(sid * ZROWS, ZROWS)])
        pltpu.sync_copy(den_sh.at[pl.ds(sid * ZROWS, ZROWS)],
                        den_hbm.at[cid, pl.ds(sid * ZROWS, ZROWS)])


def _make_sc_layer(dh):
    @functools.partial(
        pl.kernel,
        out_type=(
            jax.ShapeDtypeStruct((NC, N, dh), jnp.float32),
            jax.ShapeDtypeStruct((NC, N, 16), jnp.float32),
        ),
        mesh=_MESH,
        compiler_params=pltpu.CompilerParams(needs_layout_passes=False, use_tc_tiling_on_sc=False),
        scratch_types=[
            pltpu.VMEM((N,), jnp.float32),
            pltpu.VMEM((N,), jnp.float32),
            pltpu.VMEM((16,), jnp.float32),
            pltpu.VMEM((CH,), jnp.int32),
            pltpu.VMEM((CH,), jnp.int32),
            pltpu.VMEM((CH,), jnp.float32),
            pltpu.VMEM((CH, dh), jnp.float32),
            pltpu.SemaphoreType.DMA,
            pltpu.VMEM((CH,), jnp.int32),
            pltpu.VMEM((CH,), jnp.int32),
            pltpu.VMEM((CH,), jnp.float32),
            pltpu.VMEM((CH, dh), jnp.float32),
            pltpu.SemaphoreType.DMA,
            pltpu.VMEM((CH,), jnp.float32),
            pltpu.VMEM((CH, 16), jnp.float32),
            pltpu.VMEM_SHARED((N, dh), jnp.float32),
            pltpu.VMEM_SHARED((N, 16), jnp.float32),
        ],
    )
    def sc_layer(src_hbm, dst_hbm, et_hbm, s_hbm, d_hbm, m_hbm, h_hbm,
                 zacc_hbm, zden_hbm, acc_hbm, den_hbm, *scratch):
        _sc_layer_body(dh, src_hbm, dst_hbm, et_hbm, s_hbm, d_hbm, m_hbm,
                       h_hbm, zacc_hbm, zden_hbm, acc_hbm, den_hbm, *scratch)

    return sc_layer


_sc_layer32 = _make_sc_layer(32)
_sc_layer64 = _make_sc_layer(64)


def kernel(x, edge_index, edge_attr, W1, as1, ad1, We1, ae1, b1,
           W2, as2, ad2, We2, ae2, b2, W3, as3, ad3, We3, ae3, b3, W4, b4):
    z16 = jnp.zeros((N, 16), jnp.float32)
    z32 = jnp.zeros((N, 32), jnp.float32)
    z64 = jnp.zeros((N, 64), jnp.float32)

    h1, s1, d1, sdmax1, CS, src, dst = _tc_a(
        x, W1, as1, ad1, We1, ae1, We2, ae2, We3, ae3, edge_index)
    naccD, e1, e2, e3, etmaxT = _sc_pre(edge_attr.reshape(E * 4), CS, dst, z16)
    ett = (e1, e2, e3)
    etl1, etl2, etl3, etlmax = _tc_b(naccD[0], naccD[1])

    etmax = jnp.max(etmaxT.reshape(NC, NS, 3, 16), axis=(0, 1, 3))
    emax = jnp.maximum(etmax, etlmax[0])

    M1 = sdmax1[0, 0] + sdmax1[0, 1] + emax[0]
    m1 = jnp.full((16,), M1, jnp.float32)
    accD1 = _sc_layer32(src, dst, ett[0], s1, d1, m1, h1, z32, z16)

    h2, s2, d2, sdmax2 = _tc_lj(64, accD1, h1, s1, d1, etl1, M1,
                                b1, W2, as2, ad2)
    M2 = sdmax2[0, 0] + sdmax2[0, 1] + emax[1]
    m2 = jnp.full((16,), M2, jnp.float32)
    accD2 = _sc_layer64(src, dst, ett[1], s2, d2, m2, h2, z64, z16)

    h3, s3, d3, sdmax3 = _tc_lj(64, accD2, h2, s2, d2, etl2, M2,
                                b2, W3, as3, ad3)
    M3 = sdmax3[0, 0] + sdmax3[0, 1] + emax[2]
    m3 = jnp.full((16,), M3, jnp.float32)
    accD3 = _sc_layer64(src, dst, ett[2], s3, d3, m3, h3, z64, z16)

    return _tc_l4(accD3, h3, s3, d3, etl3, M3, b3, W4, b4)
